# Initial kernel scaffold; baseline (speedup 1.0000x reference)
import jax
import jax.numpy as jnp
from jax.experimental import pallas as pl


def kernel(node_pairs, adj_matrix, event_history, neighbor_data, node_embeds, W_proj, W_beta, b_beta, decay_theta, q1, q2):
    raise NotImplementedError("write your pallas kernel here")


# speedup vs baseline: 6.1201x; 6.1201x over previous
import jax
import jax.numpy as jnp
from jax.experimental import pallas as pl

_N = 1024
_D = 128
_P = 4096
_R = 3
_K = 16
_CURRENT_TIME = 200.0
_PAIR_BLK = 512


def _node_stage(ev_ref, idx_ref, E_ref, Wp_ref, Wb_ref, bb_ref, theta_ref,
                proj_ref, s_ref, alpha_ref):
    E = E_ref[...]
    proj = jnp.dot(E, Wp_ref[...], preferred_element_type=jnp.float32)
    proj_ref[...] = proj
    theta = theta_ref[0, 0]
    alpha_ref[...] = jnp.sum(
        jnp.exp(-theta * (_CURRENT_TIME - ev_ref[...]))).reshape(1, 1)
    idx = idx_ref[...]
    iota = jax.lax.broadcasted_iota(jnp.int32, (_N, _N), 1)
    acc = jnp.zeros((_N, _D), jnp.float32)
    for r in range(_R):
        P_r = jnp.dot(E, Wb_ref[r], preferred_element_type=jnp.float32)
        M = jnp.zeros((_N, _N), jnp.float32)
        for k in range(_K):
            col = idx[:, r * _K + k:r * _K + k + 1]
            M = M + (col == iota).astype(jnp.float32)
        acc = acc + jnp.dot(M, P_r, preferred_element_type=jnp.float32)
    bbar = jnp.mean(bb_ref[...], axis=0, keepdims=True)
    x = acc * (1.0 / (_R * _K)) + bbar
    eps = jax.nn.sigmoid(x)
    s_ref[...] = jnp.mean(eps, axis=1, keepdims=True)


def _pair_stage(m_ref, n_ref, proj_ref, s_ref, alpha_ref, q1_ref, q2_ref,
                out_ref):
    m = m_ref[...]
    n = n_ref[...]
    iota = jax.lax.broadcasted_iota(jnp.int32, (_PAIR_BLK, _N), 1)
    ohm = (m == iota).astype(jnp.float32)
    ohn = (n == iota).astype(jnp.float32)
    diff = jnp.dot(ohm - ohn, proj_ref[...], preferred_element_type=jnp.float32)
    d2 = jnp.sum(diff * diff, axis=1, keepdims=True)
    lamn = 0.5 * jnp.dot(ohm + ohn, s_ref[...],
                         preferred_element_type=jnp.float32)
    lam = -jnp.sqrt(d2 + 1e-12) + alpha_ref[0, 0] + lamn
    y = q1_ref[0, 0] * jnp.exp(lam) + q2_ref[0, 0] * lam
    out_ref[...] = jax.nn.sigmoid(y)


def kernel(node_pairs, adj_matrix, event_history, neighbor_data, node_embeds,
           W_proj, W_beta, b_beta, decay_theta, q1, q2):
    del adj_matrix
    f32 = jnp.float32
    m = node_pairs[:, 0:1].astype(jnp.int32)
    n = node_pairs[:, 1:2].astype(jnp.int32)
    T = event_history.shape[0]
    tpad = (-T) % _D
    ev = jnp.concatenate(
        [event_history.astype(f32), jnp.full((tpad,), -1e30, f32)])
    ev = ev.reshape(-1, _D)
    idx2d = neighbor_data.reshape(_N, _R * _K).astype(jnp.int32)
    theta = jnp.reshape(decay_theta.astype(f32), (1, 1))
    q1r = jnp.reshape(jnp.asarray(q1, f32), (1, 1))
    q2r = jnp.reshape(jnp.asarray(q2, f32), (1, 1))

    proj, s, alpha = pl.pallas_call(
        _node_stage,
        out_shape=[
            jax.ShapeDtypeStruct((_N, _D), f32),
            jax.ShapeDtypeStruct((_N, 1), f32),
            jax.ShapeDtypeStruct((1, 1), f32),
        ],
    )(ev, idx2d, node_embeds.astype(f32), W_proj.astype(f32),
      W_beta.astype(f32), b_beta.astype(f32), theta)

    nblk = _P // _PAIR_BLK
    out = pl.pallas_call(
        _pair_stage,
        grid=(nblk,),
        in_specs=[
            pl.BlockSpec((_PAIR_BLK, 1), lambda i: (i, 0)),
            pl.BlockSpec((_PAIR_BLK, 1), lambda i: (i, 0)),
            pl.BlockSpec((_N, _D), lambda i: (0, 0)),
            pl.BlockSpec((_N, 1), lambda i: (0, 0)),
            pl.BlockSpec((1, 1), lambda i: (0, 0)),
            pl.BlockSpec((1, 1), lambda i: (0, 0)),
            pl.BlockSpec((1, 1), lambda i: (0, 0)),
        ],
        out_specs=pl.BlockSpec((_PAIR_BLK, 1), lambda i: (i, 0)),
        out_shape=jax.ShapeDtypeStruct((_P, 1), f32),
    )(m, n, proj, s, alpha, q1r, q2r)
    return out.reshape(_P)
